# baseline (speedup 1.0000x reference)
import functools

import jax
import jax.numpy as jnp
from jax import lax
from jax.experimental import pallas as pl
from jax.experimental.pallas import tpu as pltpu
from jax.experimental.pallas import tpu_sc as plsc

N = 100000
D = 64
E = 500000
NEG_SLOPE = 0.2

NC, NS, L = 2, 16, 16
NW = NC * NS

N_PAD = 100352
PADV = N_PAD - 1
RB = 512
NBLK = N_PAD // RB

RSIZE = 10240
NRANGE = 10
NRC = NRANGE // NC
RPT = RSIZE // NS
CAP = 3968
CHUNK = 128
SCH = 64

EPT = 15872
E_PAD = EPT * NW
CH = 992
NCH = EPT // CH

_mesh = plsc.VectorSubcoreMesh(core_axis_name="c", subcore_axis_name="s",
                               num_cores=NC, num_subcores=NS)
_f32 = jnp.float32
_i32 = jnp.int32


@functools.partial(
    pl.kernel,
    out_type=jax.ShapeDtypeStruct((NW * N_PAD,), _f32),
    mesh=_mesh,
    compiler_params=pltpu.CompilerParams(needs_layout_passes=False),
    scratch_types=[
        pltpu.VMEM((N_PAD,), _f32),
        pltpu.VMEM((CH,), _i32),
    ],
)
def _hist_sc(users_hbm, pos_hbm, out_hbm, hist_v, ids_v):
    c = lax.axis_index("c")
    s = lax.axis_index("s")
    p = c * NS + s
    zero16 = jnp.zeros((L,), _f32)
    ones16 = jnp.ones((L,), _f32)

    def _z(i, _):
        hist_v[pl.ds(i * L, L)] = zero16
        return 0
    lax.fori_loop(0, N_PAD // L, _z, 0)

    base = p * EPT

    def _chunk(k, _):
        off = base + k * CH

        def _scan(j, _):
            idx = ids_v[pl.ds(j * L, L)]
            plsc.addupdate_scatter(hist_v, [idx], ones16)
            return 0

        pltpu.sync_copy(users_hbm.at[pl.ds(off, CH)], ids_v)
        lax.fori_loop(0, CH // L, _scan, 0)
        pltpu.sync_copy(pos_hbm.at[pl.ds(off, CH)], ids_v)
        lax.fori_loop(0, CH // L, _scan, 0)
        return 0

    lax.fori_loop(0, NCH, _chunk, 0)

    pltpu.sync_copy(hist_v, out_hbm.at[pl.ds(p * N_PAD, N_PAD)])


@functools.partial(
    pl.kernel,
    out_type=(jax.ShapeDtypeStruct((NRANGE * NW * CAP,), _i32),
              jax.ShapeDtypeStruct((NRANGE * NW * CAP,), _i32),
              jax.ShapeDtypeStruct((NW * L,), _i32)),
    mesh=_mesh,
    compiler_params=pltpu.CompilerParams(needs_layout_passes=False),
    scratch_types=[
        pltpu.VMEM((NRANGE * CAP,), _i32),
        pltpu.VMEM((NRANGE * CAP,), _i32),
        pltpu.VMEM((CH,), _i32),
        pltpu.VMEM((CH,), _i32),
        pltpu.VMEM((L,), _i32),
    ],
)
def _bucket_sc(users_hbm, pos_hbm, bsrc_hbm, bdst_hbm, cnt_hbm,
               st_src, st_dst, u_v, p_v, cnt_v):
    c = lax.axis_index("c")
    s = lax.axis_index("s")
    p = c * NS + s
    lane = lax.iota(_i32, L)
    padv16 = jnp.full((L,), PADV, _i32)
    zero16 = jnp.zeros((L,), _i32)

    def _fill(i, _):
        st_src[pl.ds(i * L, L)] = padv16
        st_dst[pl.ds(i * L, L)] = zero16
        return 0
    lax.fori_loop(0, NRANGE * CAP // L, _fill, 0)

    base = p * EPT

    def _append(r, cr, src_vec, dst_vec, m):
        plsc.store_compressed(st_src.at[pl.ds(r * CAP + cr, L)], src_vec,
                              mask=m)
        plsc.store_compressed(st_dst.at[pl.ds(r * CAP + cr, L)],
                              dst_vec - r * RSIZE, mask=m)
        return jnp.minimum(cr + jnp.sum(m.astype(_i32)), CAP - L)

    def _chunk(k, counts):
        off = base + k * CH
        pltpu.sync_copy(users_hbm.at[pl.ds(off, CH)], u_v)
        pltpu.sync_copy(pos_hbm.at[pl.ds(off, CH)], p_v)

        def _scan(j, counts):
            counts = list(counts)
            uv = u_v[pl.ds(j * L, L)]
            pv = p_v[pl.ds(j * L, L)]
            for r in (4, 5, 6, 7, 8, 9):
                m = (pv >= r * RSIZE) & (pv < (r + 1) * RSIZE)
                counts[r] = _append(r, counts[r], uv, pv, m)
            for r in (0, 1, 2, 3, 4):
                m = (uv >= r * RSIZE) & (uv < (r + 1) * RSIZE)
                counts[r] = _append(r, counts[r], pv, uv, m)
            return tuple(counts)

        return lax.fori_loop(0, CH // L, _scan, counts)

    counts = lax.fori_loop(0, NCH, _chunk,
                           tuple(jnp.zeros((), _i32) for _ in range(NRANGE)))

    for r in range(NRANGE):
        pltpu.sync_copy(st_src.at[pl.ds(r * CAP, CAP)],
                        bsrc_hbm.at[pl.ds((r * NW + p) * CAP, CAP)])
        pltpu.sync_copy(st_dst.at[pl.ds(r * CAP, CAP)],
                        bdst_hbm.at[pl.ds((r * NW + p) * CAP, CAP)])

    cvec = jnp.zeros((L,), _i32)
    for r in range(NRANGE):
        cvec = jnp.where(lane == r, counts[r], cvec)
    cnt_v[pl.ds(0, L)] = cvec
    pltpu.sync_copy(cnt_v, cnt_hbm.at[pl.ds(p * L, L)])


CPB = CAP // SCH


@functools.partial(
    pl.kernel,
    out_type=jax.ShapeDtypeStruct((NRANGE * RSIZE, 2 * D), _f32),
    mesh=_mesh,
    compiler_params=pltpu.CompilerParams(needs_layout_passes=False,
                                         use_tc_tiling_on_sc=False),
    scratch_types=[
        pltpu.VMEM_SHARED((RSIZE, 2 * D), _f32),
        pltpu.VMEM((2 * CPB, SCH), _i32),
        pltpu.VMEM((2 * CPB, SCH), _i32),
        pltpu.VMEM((SCH, 2 * D), _f32),
        pltpu.VMEM((SCH, 2 * D), _f32),
        pltpu.VMEM((SCH, 2 * D), _f32),
        pltpu.VMEM((SCH, 2 * D), _f32),
        pltpu.VMEM((L,), _i32),
        pltpu.VMEM((L,), _i32),
        pltpu.SemaphoreType.DMA,
        pltpu.SemaphoreType.DMA,
        pltpu.SemaphoreType.DMA,
        pltpu.SemaphoreType.DMA,
    ],
)
def _spmm_sc(y_hbm, bsrc_hbm, bdst_hbm, cnt_hbm, s_hbm,
             acc_sh, sidx_v, didx_v, rows_0, rows_1, rows_2, rows_3,
             cra_v, crb_v, gsem, psem, zsem, ssem):
    c = lax.axis_index("c")
    s = lax.axis_index("s")
    lane = lax.iota(_i32, L)
    zero16 = jnp.zeros((L,), _f32)

    def _z(i, _):
        for jj in range(2 * D // L):
            rows_0[i, pl.ds(jj * L, L)] = zero16
        return 0

    pltpu.sync_copy(cnt_hbm.at[pl.ds((2 * s) * L, L)], cra_v)
    pltpu.sync_copy(cnt_hbm.at[pl.ds((2 * s + 1) * L, L)], crb_v)
    cra = cra_v[pl.ds(0, L)]
    crb = crb_v[pl.ds(0, L)]

    for j in range(NRC):
        r = 2 * j + c

        lax.fori_loop(0, SCH, _z, 0)

        def _zacc_args(i):
            return (rows_0, acc_sh.at[pl.ds(s * RPT + i * SCH, SCH)], zsem)
        for i in range(RPT // SCH):
            pltpu.async_copy(*_zacc_args(i))

        rowa = (r * NW + 2 * s) * CPB
        rowb = (r * NW + 2 * s + 1) * CPB
        pltpu.async_copy(bsrc_hbm.at[pl.ds(rowa, CPB)],
                         sidx_v.at[pl.ds(0, CPB)], psem)
        pltpu.async_copy(bsrc_hbm.at[pl.ds(rowb, CPB)],
                         sidx_v.at[pl.ds(CPB, CPB)], psem)
        pltpu.async_copy(bdst_hbm.at[pl.ds(rowa, CPB)],
                         didx_v.at[pl.ds(0, CPB)], psem)
        pltpu.async_copy(bdst_hbm.at[pl.ds(rowb, CPB)],
                         didx_v.at[pl.ds(CPB, CPB)], psem)

        ca = jnp.sum(jnp.where(lane == r, cra, 0))
        cb = jnp.sum(jnp.where(lane == r, crb, 0))
        nch_a = (ca + SCH - 1) // SCH
        nch = nch_a + (cb + SCH - 1) // SCH

        for i in range(RPT // SCH):
            pltpu.make_async_copy(*_zacc_args(i)).wait()
        pltpu.make_async_copy(bsrc_hbm.at[pl.ds(rowa, CPB)],
                              sidx_v.at[pl.ds(0, CPB)], psem).wait()
        pltpu.make_async_copy(bsrc_hbm.at[pl.ds(rowb, CPB)],
                              sidx_v.at[pl.ds(CPB, CPB)], psem).wait()
        pltpu.make_async_copy(bdst_hbm.at[pl.ds(rowa, CPB)],
                              didx_v.at[pl.ds(0, CPB)], psem).wait()
        pltpu.make_async_copy(bdst_hbm.at[pl.ds(rowb, CPB)],
                              didx_v.at[pl.ds(CPB, CPB)], psem).wait()
        plsc.subcore_barrier()

        def _row_of(k):
            return jnp.where(k < nch_a, k, k - nch_a + CPB)

        def _fire(k, buf):
            pltpu.async_copy(y_hbm.at[sidx_v.at[_row_of(k)]], buf, gsem)

        def _wait(k, buf):
            pltpu.make_async_copy(y_hbm.at[sidx_v.at[_row_of(k)]], buf,
                                  gsem).wait()

        def _scat(k, buf):
            pltpu.sync_copy(buf, acc_sh.at[didx_v.at[_row_of(k)]], add=True)

        def _wait_scat(buf):
            pass

        bufs = (rows_0, rows_1, rows_2, rows_3)
        for i in range(4):
            @pl.when(i < nch)
            def _(i=i):
                _fire(i, bufs[i])

        def _quad(q, _):
            k4 = 4 * q
            for i in range(4):
                k = k4 + i

                @pl.when(k < nch)
                def _(k=k, i=i):
                    _wait(k, bufs[i])
                    _scat(k, bufs[i])

                @pl.when(k + 4 < nch)
                def _(k=k, i=i):
                    _wait_scat(bufs[i])
                    _fire(k + 4, bufs[i])
            return 0

        lax.fori_loop(0, (nch + 3) // 4, _quad, 0)
        for i in range(4):
            @pl.when(i < jnp.minimum(nch, 4))
            def _(i=i):
                _wait_scat(bufs[i])

        plsc.subcore_barrier()
        pltpu.sync_copy(acc_sh.at[pl.ds(s * RPT, RPT)],
                        s_hbm.at[pl.ds(r * RSIZE + s * RPT, RPT)])
        plsc.subcore_barrier()


KPT = EPT // CHUNK
ERWS = E_PAD // CHUNK


@functools.partial(
    pl.kernel,
    out_type=(jax.ShapeDtypeStruct((ERWS, CHUNK), _f32),
              jax.ShapeDtypeStruct((ERWS, CHUNK), _f32)),
    mesh=_mesh,
    compiler_params=pltpu.CompilerParams(needs_layout_passes=False,
                                         use_tc_tiling_on_sc=False),
    scratch_types=[
        pltpu.VMEM((KPT, CHUNK), _i32),
        pltpu.VMEM((KPT, CHUNK), _i32),
        pltpu.VMEM((KPT, CHUNK), _i32),
        pltpu.VMEM((CHUNK, D), _f32),
        pltpu.VMEM((CHUNK, D), _f32),
        pltpu.VMEM((CHUNK, D), _f32),
        pltpu.VMEM((CHUNK, D), _f32),
        pltpu.VMEM((CHUNK, D), _f32),
        pltpu.VMEM((CHUNK, D), _f32),
        pltpu.VMEM((KPT, CHUNK), _f32),
        pltpu.VMEM((KPT, CHUNK), _f32),
        pltpu.SemaphoreType.DMA,
        pltpu.SemaphoreType.DMA,
    ],
)
def _sim_sc(ef_hbm, u_hbm, p_hbm, n_hbm, psim_hbm, nsim_hbm,
            uid_v, pid_v, nid_v, ur_a, pr_a, nr_a, ur_b, pr_b, nr_b,
            po_all, no_all, gsem, isem):
    c = lax.axis_index("c")
    s = lax.axis_index("s")
    w = c * NS + s
    lane = lax.iota(_i32, L)
    rbase = w * KPT

    pltpu.async_copy(u_hbm.at[pl.ds(rbase, KPT)], uid_v, isem)
    pltpu.async_copy(p_hbm.at[pl.ds(rbase, KPT)], pid_v, isem)
    pltpu.async_copy(n_hbm.at[pl.ds(rbase, KPT)], nid_v, isem)
    pltpu.make_async_copy(u_hbm.at[pl.ds(rbase, KPT)], uid_v, isem).wait()
    pltpu.make_async_copy(p_hbm.at[pl.ds(rbase, KPT)], pid_v, isem).wait()
    pltpu.make_async_copy(n_hbm.at[pl.ds(rbase, KPT)], nid_v, isem).wait()

    def _fire(k, ur, pr, nr):
        pltpu.async_copy(ef_hbm.at[uid_v.at[k]], ur, gsem)
        pltpu.async_copy(ef_hbm.at[pid_v.at[k]], pr, gsem)
        pltpu.async_copy(ef_hbm.at[nid_v.at[k]], nr, gsem)

    def _waitg(k, ur, pr, nr):
        pltpu.make_async_copy(ef_hbm.at[uid_v.at[k]], ur, gsem).wait()
        pltpu.make_async_copy(ef_hbm.at[pid_v.at[k]], pr, gsem).wait()
        pltpu.make_async_copy(ef_hbm.at[nid_v.at[k]], nr, gsem).wait()

    def _compute(k, ur, pr, nr):
        for g in range(CHUNK // L):
            riv = lane + g * L

            def _dstep(t, carry):
                pacc, nacc = carry
                for dd in range(8):
                    col = jnp.full((L,), t * 8 + dd, _i32)
                    uv = plsc.load_gather(ur, [riv, col])
                    pv = plsc.load_gather(pr, [riv, col])
                    nv = plsc.load_gather(nr, [riv, col])
                    pacc = pacc + uv * pv
                    nacc = nacc + uv * nv
                return (pacc, nacc)

            pacc, nacc = lax.fori_loop(
                0, D // 8, _dstep,
                (jnp.zeros((L,), _f32), jnp.zeros((L,), _f32)))
            po_all[k, pl.ds(g * L, L)] = pacc * (1.0 / 16.0)
            no_all[k, pl.ds(g * L, L)] = nacc * (1.0 / 16.0)

    _fire(0, ur_a, pr_a, nr_a)

    def _pair(k2, _):
        k0 = 2 * k2
        _fire(k0 + 1, ur_b, pr_b, nr_b)
        _waitg(k0, ur_a, pr_a, nr_a)
        _compute(k0, ur_a, pr_a, nr_a)

        @pl.when(k0 + 2 < KPT)
        def _():
            _fire(k0 + 2, ur_a, pr_a, nr_a)
        _waitg(k0 + 1, ur_b, pr_b, nr_b)
        _compute(k0 + 1, ur_b, pr_b, nr_b)
        return 0

    lax.fori_loop(0, KPT // 2, _pair, 0)
    pltpu.sync_copy(po_all, psim_hbm.at[pl.ds(rbase, KPT)])
    pltpu.sync_copy(no_all, nsim_hbm.at[pl.ds(rbase, KPT)])


def _norms(h):
    deg = jnp.sum(h, axis=1, keepdims=True)
    d1 = jnp.where(deg > 0, lax.rsqrt(jnp.maximum(deg, 1e-30)), 0.0)
    d2 = lax.rsqrt(deg + 1.0)
    return d1, d2


def _make_y(e, w1, w2, d1, d2):
    dn = (((1,), (1,)), ((), ()))
    x1 = lax.dot_general(e, w1, dn, preferred_element_type=_f32)
    x2 = lax.dot_general(e, w2, dn, preferred_element_type=_f32)
    return jnp.concatenate([d2 * x1, d1 * (e * x2)], axis=1)


def _layer_e(sv, yv, d1, d2):
    h = (d2 * (sv[:, :D] + yv[:, :D]) + d1 * (sv[:, D:] + yv[:, D:]))
    h = jnp.where(h >= 0, h, NEG_SLOPE * h)
    nr = jnp.sqrt(jnp.sum(h * h, axis=1, keepdims=True))
    return h / jnp.maximum(nr, 1e-12)


def _pre1_body(ht_ref, e_ref, w1_ref, w2_ref, y_ref, d1_ref, d2_ref):
    d1, d2 = _norms(ht_ref[...])
    d1_ref[...] = d1
    d2_ref[...] = d2
    y_ref[...] = _make_y(e_ref[...], w1_ref[...], w2_ref[...], d1, d2)


def _pre1_tc(hist_t, emb, w1, w2):
    return pl.pallas_call(
        _pre1_body,
        grid=(NBLK,),
        in_specs=[pl.BlockSpec((RB, NW), lambda i: (i, 0)),
                  pl.BlockSpec((RB, D), lambda i: (i, 0)),
                  pl.BlockSpec((D, D), lambda i: (0, 0)),
                  pl.BlockSpec((D, D), lambda i: (0, 0))],
        out_specs=(pl.BlockSpec((RB, 2 * D), lambda i: (i, 0)),
                   pl.BlockSpec((RB, 1), lambda i: (i, 0)),
                   pl.BlockSpec((RB, 1), lambda i: (i, 0))),
        out_shape=(jax.ShapeDtypeStruct((N_PAD, 2 * D), _f32),
                   jax.ShapeDtypeStruct((N_PAD, 1), _f32),
                   jax.ShapeDtypeStruct((N_PAD, 1), _f32)),
    )(hist_t, emb, w1, w2)


def _mid_body(s_ref, y_ref, d1_ref, d2_ref, a_ref, w1_ref, w2_ref,
              yn_ref, ao_ref):
    d1 = d1_ref[...]
    d2 = d2_ref[...]
    e = _layer_e(s_ref[...], y_ref[...], d1, d2)
    ao_ref[...] = a_ref[...] + e
    yn_ref[...] = _make_y(e, w1_ref[...], w2_ref[...], d1, d2)


def _mid_tc(s, y, d1, d2, acc, w1, w2):
    return pl.pallas_call(
        _mid_body,
        grid=(NBLK,),
        in_specs=[pl.BlockSpec((RB, 2 * D), lambda i: (i, 0)),
                  pl.BlockSpec((RB, 2 * D), lambda i: (i, 0)),
                  pl.BlockSpec((RB, 1), lambda i: (i, 0)),
                  pl.BlockSpec((RB, 1), lambda i: (i, 0)),
                  pl.BlockSpec((RB, D), lambda i: (i, 0)),
                  pl.BlockSpec((D, D), lambda i: (0, 0)),
                  pl.BlockSpec((D, D), lambda i: (0, 0))],
        out_specs=(pl.BlockSpec((RB, 2 * D), lambda i: (i, 0)),
                   pl.BlockSpec((RB, D), lambda i: (i, 0))),
        out_shape=(jax.ShapeDtypeStruct((N_PAD, 2 * D), _f32),
                   jax.ShapeDtypeStruct((N_PAD, D), _f32)),
    )(s, y, d1, d2, acc, w1, w2)


def _postf_body(s_ref, y_ref, d1_ref, d2_ref, a_ref, ao_ref):
    e = _layer_e(s_ref[...], y_ref[...], d1_ref[...], d2_ref[...])
    ao_ref[...] = a_ref[...] + e


def _postf_tc(s, y, d1, d2, acc):
    return pl.pallas_call(
        _postf_body,
        grid=(NBLK,),
        in_specs=[pl.BlockSpec((RB, 2 * D), lambda i: (i, 0)),
                  pl.BlockSpec((RB, 2 * D), lambda i: (i, 0)),
                  pl.BlockSpec((RB, 1), lambda i: (i, 0)),
                  pl.BlockSpec((RB, 1), lambda i: (i, 0)),
                  pl.BlockSpec((RB, D), lambda i: (i, 0))],
        out_specs=pl.BlockSpec((RB, D), lambda i: (i, 0)),
        out_shape=jax.ShapeDtypeStruct((N_PAD, D), _f32),
    )(s, y, d1, d2, acc)


def kernel(users, pos, neg, Emb, W1_1, W2_1, W1_2, W2_2, W1_3, W2_3):
    users_p = jnp.pad(users.astype(_i32), (0, E_PAD - E),
                      constant_values=PADV)
    pos_p = jnp.pad(pos.astype(_i32), (0, E_PAD - E), constant_values=PADV)
    neg_p = jnp.pad(neg.astype(_i32), (0, E_PAD - E), constant_values=PADV)
    emb_p = jnp.pad(Emb, ((0, N_PAD - N), (0, 0)))

    hist = _hist_sc(users_p, pos_p).reshape(NW, N_PAD)
    bsrc, bdst, cnt = _bucket_sc(users_p, pos_p)
    bsrc2 = bsrc.reshape(NRANGE * NW * CPB, SCH)
    bdst2 = bdst.reshape(NRANGE * NW * CPB, SCH)

    y1, d1, d2 = _pre1_tc(hist.T, emb_p, W1_1, W2_1)
    s1 = _spmm_sc(y1, bsrc2, bdst2, cnt)
    y2, acc1 = _mid_tc(s1[:N_PAD], y1, d1, d2, emb_p, W1_2, W2_2)
    s2 = _spmm_sc(y2, bsrc2, bdst2, cnt)
    y3, acc2 = _mid_tc(s2[:N_PAD], y2, d1, d2, acc1, W1_3, W2_3)
    s3 = _spmm_sc(y3, bsrc2, bdst2, cnt)
    acc = _postf_tc(s3[:N_PAD], y3, d1, d2, acc2)

    psim, nsim = _sim_sc(acc, users_p.reshape(ERWS, CHUNK),
                         pos_p.reshape(ERWS, CHUNK),
                         neg_p.reshape(ERWS, CHUNK))
    return psim.reshape(E_PAD)[:E], nsim.reshape(E_PAD)[:E]


# speedup vs baseline: 1.0088x; 1.0088x over previous
import functools

import jax
import jax.numpy as jnp
from jax import lax
from jax.experimental import pallas as pl
from jax.experimental.pallas import tpu as pltpu
from jax.experimental.pallas import tpu_sc as plsc

N = 100000
D = 64
E = 500000
NEG_SLOPE = 0.2

NC, NS, L = 2, 16, 16
NW = NC * NS

N_PAD = 100352
PADV = N_PAD - 1
RB = 512
NBLK = N_PAD // RB

RSIZE = 10240
NRANGE = 10
NRC = NRANGE // NC
RPT = RSIZE // NS
CAP = 3968
CHUNK = 128
SCH = 64

EPT = 15872
E_PAD = EPT * NW
CH = 992
NCH = EPT // CH

_mesh = plsc.VectorSubcoreMesh(core_axis_name="c", subcore_axis_name="s",
                               num_cores=NC, num_subcores=NS)
_f32 = jnp.float32
_i32 = jnp.int32


@functools.partial(
    pl.kernel,
    out_type=jax.ShapeDtypeStruct((NW * N_PAD,), _f32),
    mesh=_mesh,
    compiler_params=pltpu.CompilerParams(needs_layout_passes=False),
    scratch_types=[
        pltpu.VMEM((N_PAD,), _f32),
        pltpu.VMEM((CH,), _i32),
    ],
)
def _hist_sc(users_hbm, pos_hbm, out_hbm, hist_v, ids_v):
    c = lax.axis_index("c")
    s = lax.axis_index("s")
    p = c * NS + s
    zero16 = jnp.zeros((L,), _f32)
    ones16 = jnp.ones((L,), _f32)

    def _z(i, _):
        hist_v[pl.ds(i * L, L)] = zero16
        return 0
    lax.fori_loop(0, N_PAD // L, _z, 0)

    base = p * EPT

    def _chunk(k, _):
        off = base + k * CH

        def _scan(j, _):
            idx = ids_v[pl.ds(j * L, L)]
            plsc.addupdate_scatter(hist_v, [idx], ones16)
            return 0

        pltpu.sync_copy(users_hbm.at[pl.ds(off, CH)], ids_v)
        lax.fori_loop(0, CH // L, _scan, 0)
        pltpu.sync_copy(pos_hbm.at[pl.ds(off, CH)], ids_v)
        lax.fori_loop(0, CH // L, _scan, 0)
        return 0

    lax.fori_loop(0, NCH, _chunk, 0)

    pltpu.sync_copy(hist_v, out_hbm.at[pl.ds(p * N_PAD, N_PAD)])


@functools.partial(
    pl.kernel,
    out_type=(jax.ShapeDtypeStruct((NRANGE * NW * CAP,), _i32),
              jax.ShapeDtypeStruct((NRANGE * NW * CAP,), _i32),
              jax.ShapeDtypeStruct((NW * L,), _i32)),
    mesh=_mesh,
    compiler_params=pltpu.CompilerParams(needs_layout_passes=False),
    scratch_types=[
        pltpu.VMEM((NRANGE * CAP,), _i32),
        pltpu.VMEM((NRANGE * CAP,), _i32),
        pltpu.VMEM((CH,), _i32),
        pltpu.VMEM((CH,), _i32),
        pltpu.VMEM((L,), _i32),
    ],
)
def _bucket_sc(users_hbm, pos_hbm, bsrc_hbm, bdst_hbm, cnt_hbm,
               st_src, st_dst, u_v, p_v, cnt_v):
    c = lax.axis_index("c")
    s = lax.axis_index("s")
    p = c * NS + s
    lane = lax.iota(_i32, L)
    padv16 = jnp.full((L,), PADV, _i32)
    zero16 = jnp.zeros((L,), _i32)

    def _fill(i, _):
        st_src[pl.ds(i * L, L)] = padv16
        st_dst[pl.ds(i * L, L)] = zero16
        return 0
    lax.fori_loop(0, NRANGE * CAP // L, _fill, 0)

    base = p * EPT

    def _append(r, cr, src_vec, dst_vec, m):
        plsc.store_compressed(st_src.at[pl.ds(r * CAP + cr, L)], src_vec,
                              mask=m)
        plsc.store_compressed(st_dst.at[pl.ds(r * CAP + cr, L)],
                              dst_vec - r * RSIZE, mask=m)
        return jnp.minimum(cr + jnp.sum(m.astype(_i32)), CAP - L)

    def _chunk(k, counts):
        off = base + k * CH
        pltpu.sync_copy(users_hbm.at[pl.ds(off, CH)], u_v)
        pltpu.sync_copy(pos_hbm.at[pl.ds(off, CH)], p_v)

        def _scan(j, counts):
            counts = list(counts)
            uv = u_v[pl.ds(j * L, L)]
            pv = p_v[pl.ds(j * L, L)]
            for r in (4, 5, 6, 7, 8, 9):
                m = (pv >= r * RSIZE) & (pv < (r + 1) * RSIZE)
                counts[r] = _append(r, counts[r], uv, pv, m)
            for r in (0, 1, 2, 3, 4):
                m = (uv >= r * RSIZE) & (uv < (r + 1) * RSIZE)
                counts[r] = _append(r, counts[r], pv, uv, m)
            return tuple(counts)

        return lax.fori_loop(0, CH // L, _scan, counts)

    counts = lax.fori_loop(0, NCH, _chunk,
                           tuple(jnp.zeros((), _i32) for _ in range(NRANGE)))

    for r in range(NRANGE):
        pltpu.sync_copy(st_src.at[pl.ds(r * CAP, CAP)],
                        bsrc_hbm.at[pl.ds((r * NW + p) * CAP, CAP)])
        pltpu.sync_copy(st_dst.at[pl.ds(r * CAP, CAP)],
                        bdst_hbm.at[pl.ds((r * NW + p) * CAP, CAP)])

    cvec = jnp.zeros((L,), _i32)
    for r in range(NRANGE):
        cvec = jnp.where(lane == r, counts[r], cvec)
    cnt_v[pl.ds(0, L)] = cvec
    pltpu.sync_copy(cnt_v, cnt_hbm.at[pl.ds(p * L, L)])


CPB = CAP // SCH


@functools.partial(
    pl.kernel,
    out_type=jax.ShapeDtypeStruct((NRANGE * RSIZE, 2 * D), _f32),
    mesh=_mesh,
    compiler_params=pltpu.CompilerParams(needs_layout_passes=False,
                                         use_tc_tiling_on_sc=False),
    scratch_types=[
        pltpu.VMEM_SHARED((RSIZE, 2 * D), _f32),
        pltpu.VMEM((2 * CPB, SCH), _i32),
        pltpu.VMEM((2 * CPB, SCH), _i32),
        pltpu.VMEM((SCH, 2 * D), _f32),
        pltpu.VMEM((SCH, 2 * D), _f32),
        pltpu.VMEM((SCH, 2 * D), _f32),
        pltpu.VMEM((SCH, 2 * D), _f32),
        pltpu.VMEM((L,), _i32),
        pltpu.VMEM((L,), _i32),
        pltpu.SemaphoreType.DMA,
        pltpu.SemaphoreType.DMA,
        pltpu.SemaphoreType.DMA,
        pltpu.SemaphoreType.DMA,
    ],
)
def _spmm_sc(y_hbm, bsrc_hbm, bdst_hbm, cnt_hbm, s_hbm,
             acc_sh, sidx_v, didx_v, rows_0, rows_1, rows_2, rows_3,
             cra_v, crb_v, gsem, psem, zsem, ssem):
    c = lax.axis_index("c")
    s = lax.axis_index("s")
    lane = lax.iota(_i32, L)
    zero16 = jnp.zeros((L,), _f32)

    def _z(i, _):
        for jj in range(2 * D // L):
            rows_0[i, pl.ds(jj * L, L)] = zero16
        return 0

    pltpu.sync_copy(cnt_hbm.at[pl.ds((2 * s) * L, L)], cra_v)
    pltpu.sync_copy(cnt_hbm.at[pl.ds((2 * s + 1) * L, L)], crb_v)
    cra = cra_v[pl.ds(0, L)]
    crb = crb_v[pl.ds(0, L)]

    for j in range(NRC):
        r = 2 * j + c

        lax.fori_loop(0, SCH, _z, 0)

        def _zacc_args(i):
            return (rows_0, acc_sh.at[pl.ds(s * RPT + i * SCH, SCH)], zsem)
        for i in range(RPT // SCH):
            pltpu.async_copy(*_zacc_args(i))

        rowa = (r * NW + 2 * s) * CPB
        rowb = (r * NW + 2 * s + 1) * CPB
        pltpu.async_copy(bsrc_hbm.at[pl.ds(rowa, CPB)],
                         sidx_v.at[pl.ds(0, CPB)], psem)
        pltpu.async_copy(bsrc_hbm.at[pl.ds(rowb, CPB)],
                         sidx_v.at[pl.ds(CPB, CPB)], psem)
        pltpu.async_copy(bdst_hbm.at[pl.ds(rowa, CPB)],
                         didx_v.at[pl.ds(0, CPB)], psem)
        pltpu.async_copy(bdst_hbm.at[pl.ds(rowb, CPB)],
                         didx_v.at[pl.ds(CPB, CPB)], psem)

        ca = jnp.sum(jnp.where(lane == r, cra, 0))
        cb = jnp.sum(jnp.where(lane == r, crb, 0))
        nch_a = (ca + SCH - 1) // SCH
        nch = nch_a + (cb + SCH - 1) // SCH

        for i in range(RPT // SCH):
            pltpu.make_async_copy(*_zacc_args(i)).wait()
        pltpu.make_async_copy(bsrc_hbm.at[pl.ds(rowa, CPB)],
                              sidx_v.at[pl.ds(0, CPB)], psem).wait()
        pltpu.make_async_copy(bsrc_hbm.at[pl.ds(rowb, CPB)],
                              sidx_v.at[pl.ds(CPB, CPB)], psem).wait()
        pltpu.make_async_copy(bdst_hbm.at[pl.ds(rowa, CPB)],
                              didx_v.at[pl.ds(0, CPB)], psem).wait()
        pltpu.make_async_copy(bdst_hbm.at[pl.ds(rowb, CPB)],
                              didx_v.at[pl.ds(CPB, CPB)], psem).wait()
        plsc.subcore_barrier()

        def _row_of(k):
            return jnp.where(k < nch_a, k, k - nch_a + CPB)

        def _fire(k, buf):
            pltpu.async_copy(y_hbm.at[sidx_v.at[_row_of(k)]], buf, gsem)

        def _wait(k, buf):
            pltpu.make_async_copy(y_hbm.at[sidx_v.at[_row_of(k)]], buf,
                                  gsem).wait()

        def _scat(k, buf):
            pltpu.sync_copy(buf, acc_sh.at[didx_v.at[_row_of(k)]], add=True)

        def _wait_scat(buf):
            pass

        bufs = (rows_0, rows_1, rows_2, rows_3)
        for i in range(4):
            @pl.when(i < nch)
            def _(i=i):
                _fire(i, bufs[i])

        def _quad(q, _):
            k4 = 4 * q
            for i in range(4):
                k = k4 + i

                @pl.when(k < nch)
                def _(k=k, i=i):
                    _wait(k, bufs[i])
                    _scat(k, bufs[i])

                @pl.when(k + 4 < nch)
                def _(k=k, i=i):
                    _wait_scat(bufs[i])
                    _fire(k + 4, bufs[i])
            return 0

        lax.fori_loop(0, (nch + 3) // 4, _quad, 0)
        for i in range(4):
            @pl.when(i < jnp.minimum(nch, 4))
            def _(i=i):
                _wait_scat(bufs[i])

        plsc.subcore_barrier()
        pltpu.sync_copy(acc_sh.at[pl.ds(s * RPT, RPT)],
                        s_hbm.at[pl.ds(r * RSIZE + s * RPT, RPT)])
        plsc.subcore_barrier()


KPT = EPT // CHUNK
ERWS = E_PAD // CHUNK


@functools.partial(
    pl.kernel,
    out_type=(jax.ShapeDtypeStruct((ERWS, CHUNK), _f32),
              jax.ShapeDtypeStruct((ERWS, CHUNK), _f32)),
    mesh=_mesh,
    compiler_params=pltpu.CompilerParams(needs_layout_passes=False,
                                         use_tc_tiling_on_sc=False),
    scratch_types=[
        pltpu.VMEM((KPT, CHUNK), _i32),
        pltpu.VMEM((KPT, CHUNK), _i32),
        pltpu.VMEM((KPT, CHUNK), _i32),
        pltpu.VMEM((CHUNK, D), _f32),
        pltpu.VMEM((CHUNK, D), _f32),
        pltpu.VMEM((CHUNK, D), _f32),
        pltpu.VMEM((CHUNK, D), _f32),
        pltpu.VMEM((CHUNK, D), _f32),
        pltpu.VMEM((CHUNK, D), _f32),
        pltpu.VMEM((KPT, CHUNK), _f32),
        pltpu.VMEM((KPT, CHUNK), _f32),
        pltpu.SemaphoreType.DMA,
        pltpu.SemaphoreType.DMA,
    ],
)
def _sim_sc(ef_hbm, u_hbm, p_hbm, n_hbm, psim_hbm, nsim_hbm,
            uid_v, pid_v, nid_v, ur_a, pr_a, nr_a, ur_b, pr_b, nr_b,
            po_all, no_all, gsem, isem):
    c = lax.axis_index("c")
    s = lax.axis_index("s")
    w = c * NS + s
    lane = lax.iota(_i32, L)
    rbase = w * KPT

    pltpu.async_copy(u_hbm.at[pl.ds(rbase, KPT)], uid_v, isem)
    pltpu.async_copy(p_hbm.at[pl.ds(rbase, KPT)], pid_v, isem)
    pltpu.async_copy(n_hbm.at[pl.ds(rbase, KPT)], nid_v, isem)
    pltpu.make_async_copy(u_hbm.at[pl.ds(rbase, KPT)], uid_v, isem).wait()
    pltpu.make_async_copy(p_hbm.at[pl.ds(rbase, KPT)], pid_v, isem).wait()
    pltpu.make_async_copy(n_hbm.at[pl.ds(rbase, KPT)], nid_v, isem).wait()

    def _fire(k, ur, pr, nr):
        pltpu.async_copy(ef_hbm.at[uid_v.at[k]], ur, gsem)
        pltpu.async_copy(ef_hbm.at[pid_v.at[k]], pr, gsem)
        pltpu.async_copy(ef_hbm.at[nid_v.at[k]], nr, gsem)

    def _waitg(k, ur, pr, nr):
        pltpu.make_async_copy(ef_hbm.at[uid_v.at[k]], ur, gsem).wait()
        pltpu.make_async_copy(ef_hbm.at[pid_v.at[k]], pr, gsem).wait()
        pltpu.make_async_copy(ef_hbm.at[nid_v.at[k]], nr, gsem).wait()

    def _compute(k, ur, pr, nr):
        for g in range(CHUNK // L):
            riv = lane + g * L

            def _dstep(t, carry):
                pacc, nacc = carry
                for dd in range(8):
                    col = jnp.full((L,), t * 8 + dd, _i32)
                    uv = plsc.load_gather(ur, [riv, col])
                    pv = plsc.load_gather(pr, [riv, col])
                    nv = plsc.load_gather(nr, [riv, col])
                    pacc = pacc + uv * pv
                    nacc = nacc + uv * nv
                return (pacc, nacc)

            pacc, nacc = lax.fori_loop(
                0, D // 8, _dstep,
                (jnp.zeros((L,), _f32), jnp.zeros((L,), _f32)))
            po_all[k, pl.ds(g * L, L)] = pacc * (1.0 / 16.0)
            no_all[k, pl.ds(g * L, L)] = nacc * (1.0 / 16.0)

    _fire(0, ur_a, pr_a, nr_a)

    def _pair(k2, _):
        k0 = 2 * k2
        _fire(k0 + 1, ur_b, pr_b, nr_b)
        _waitg(k0, ur_a, pr_a, nr_a)
        _compute(k0, ur_a, pr_a, nr_a)

        @pl.when(k0 + 2 < KPT)
        def _():
            _fire(k0 + 2, ur_a, pr_a, nr_a)
        _waitg(k0 + 1, ur_b, pr_b, nr_b)
        _compute(k0 + 1, ur_b, pr_b, nr_b)
        return 0

    lax.fori_loop(0, KPT // 2, _pair, 0)
    pltpu.sync_copy(po_all, psim_hbm.at[pl.ds(rbase, KPT)])
    pltpu.sync_copy(no_all, nsim_hbm.at[pl.ds(rbase, KPT)])


def _norms(h):
    deg = jnp.sum(h, axis=1, keepdims=True)
    d1 = jnp.where(deg > 0, lax.rsqrt(jnp.maximum(deg, 1e-30)), 0.0)
    d2 = lax.rsqrt(deg + 1.0)
    return d1, d2


def _make_y(e, w1, w2, d1, d2):
    dn = (((1,), (1,)), ((), ()))
    x1 = lax.dot_general(e, w1, dn, preferred_element_type=_f32)
    x2 = lax.dot_general(e, w2, dn, preferred_element_type=_f32)
    return jnp.concatenate([d2 * x1, d1 * (e * x2)], axis=1)


def _layer_e(sv, yv, d1, d2):
    h = (d2 * (sv[:, :D] + yv[:, :D]) + d1 * (sv[:, D:] + yv[:, D:]))
    h = jnp.where(h >= 0, h, NEG_SLOPE * h)
    nr = jnp.sqrt(jnp.sum(h * h, axis=1, keepdims=True))
    return h / jnp.maximum(nr, 1e-12)


def _pre1_body(ht_ref, e_ref, w1_ref, w2_ref, y_ref, d1_ref, d2_ref):
    d1, d2 = _norms(ht_ref[...])
    d1_ref[...] = d1
    d2_ref[...] = d2
    y_ref[...] = _make_y(e_ref[...], w1_ref[...], w2_ref[...], d1, d2)


def _pre1_tc(hist_t, emb, w1, w2):
    return pl.pallas_call(
        _pre1_body,
        grid=(NBLK,),
        in_specs=[pl.BlockSpec((RB, NW), lambda i: (i, 0)),
                  pl.BlockSpec((RB, D), lambda i: (i, 0)),
                  pl.BlockSpec((D, D), lambda i: (0, 0)),
                  pl.BlockSpec((D, D), lambda i: (0, 0))],
        out_specs=(pl.BlockSpec((RB, 2 * D), lambda i: (i, 0)),
                   pl.BlockSpec((RB, 1), lambda i: (i, 0)),
                   pl.BlockSpec((RB, 1), lambda i: (i, 0))),
        out_shape=(jax.ShapeDtypeStruct((N_PAD, 2 * D), _f32),
                   jax.ShapeDtypeStruct((N_PAD, 1), _f32),
                   jax.ShapeDtypeStruct((N_PAD, 1), _f32)),
    )(hist_t, emb, w1, w2)


def _mid_body(s_ref, y_ref, d1_ref, d2_ref, a_ref, w1_ref, w2_ref,
              yn_ref, ao_ref):
    d1 = d1_ref[...]
    d2 = d2_ref[...]
    e = _layer_e(s_ref[...], y_ref[...], d1, d2)
    ao_ref[...] = a_ref[...] + e
    yn_ref[...] = _make_y(e, w1_ref[...], w2_ref[...], d1, d2)


def _mid_tc(s, y, d1, d2, acc, w1, w2):
    return pl.pallas_call(
        _mid_body,
        grid=(NBLK,),
        in_specs=[pl.BlockSpec((RB, 2 * D), lambda i: (i, 0)),
                  pl.BlockSpec((RB, 2 * D), lambda i: (i, 0)),
                  pl.BlockSpec((RB, 1), lambda i: (i, 0)),
                  pl.BlockSpec((RB, 1), lambda i: (i, 0)),
                  pl.BlockSpec((RB, D), lambda i: (i, 0)),
                  pl.BlockSpec((D, D), lambda i: (0, 0)),
                  pl.BlockSpec((D, D), lambda i: (0, 0))],
        out_specs=(pl.BlockSpec((RB, 2 * D), lambda i: (i, 0)),
                   pl.BlockSpec((RB, D), lambda i: (i, 0))),
        out_shape=(jax.ShapeDtypeStruct((N_PAD, 2 * D), _f32),
                   jax.ShapeDtypeStruct((N_PAD, D), _f32)),
    )(s, y, d1, d2, acc, w1, w2)


def _postf_body(s_ref, y_ref, d1_ref, d2_ref, a_ref, ao_ref):
    e = _layer_e(s_ref[...], y_ref[...], d1_ref[...], d2_ref[...])
    ao_ref[...] = a_ref[...] + e


def _postf_tc(s, y, d1, d2, acc):
    return pl.pallas_call(
        _postf_body,
        grid=(NBLK,),
        in_specs=[pl.BlockSpec((RB, 2 * D), lambda i: (i, 0)),
                  pl.BlockSpec((RB, 2 * D), lambda i: (i, 0)),
                  pl.BlockSpec((RB, 1), lambda i: (i, 0)),
                  pl.BlockSpec((RB, 1), lambda i: (i, 0)),
                  pl.BlockSpec((RB, D), lambda i: (i, 0))],
        out_specs=pl.BlockSpec((RB, D), lambda i: (i, 0)),
        out_shape=jax.ShapeDtypeStruct((N_PAD, D), _f32),
    )(s, y, d1, d2, acc)


def kernel(users, pos, neg, Emb, W1_1, W2_1, W1_2, W2_2, W1_3, W2_3):
    users_p = jnp.pad(users.astype(_i32), (0, E_PAD - E),
                      constant_values=PADV)
    pos_p = jnp.pad(pos.astype(_i32), (0, E_PAD - E), constant_values=PADV)
    neg_p = jnp.pad(neg.astype(_i32), (0, E_PAD - E), constant_values=PADV)
    emb_p = jnp.pad(Emb, ((0, N_PAD - N), (0, 0)))

    hist = _hist_sc(users_p, pos_p).reshape(NW, N_PAD)
    bsrc, bdst, cnt = _bucket_sc(users_p, pos_p)
    bsrc2 = bsrc.reshape(NRANGE * NW * CPB, SCH)
    bdst2 = bdst.reshape(NRANGE * NW * CPB, SCH)

    y1, d1, d2 = _pre1_tc(hist.T, emb_p, W1_1, W2_1)
    s1 = _spmm_sc(y1, bsrc2, bdst2, cnt)
    y2, acc1 = _mid_tc(s1, y1, d1, d2, emb_p, W1_2, W2_2)
    s2 = _spmm_sc(y2, bsrc2, bdst2, cnt)
    y3, acc2 = _mid_tc(s2, y2, d1, d2, acc1, W1_3, W2_3)
    s3 = _spmm_sc(y3, bsrc2, bdst2, cnt)
    acc = _postf_tc(s3, y3, d1, d2, acc2)

    psim, nsim = _sim_sc(acc, users_p.reshape(ERWS, CHUNK),
                         pos_p.reshape(ERWS, CHUNK),
                         neg_p.reshape(ERWS, CHUNK))
    return psim.reshape(E_PAD)[:E], nsim.reshape(E_PAD)[:E]


# speedup vs baseline: 1.0099x; 1.0011x over previous
import functools

import jax
import jax.numpy as jnp
from jax import lax
from jax.experimental import pallas as pl
from jax.experimental.pallas import tpu as pltpu
from jax.experimental.pallas import tpu_sc as plsc

N = 100000
D = 64
E = 500000
NEG_SLOPE = 0.2

NC, NS, L = 2, 16, 16
NW = NC * NS

N_PAD = 100352
PADV = N_PAD - 1
RB = 512
NBLK = N_PAD // RB

RSIZE = 10240
NRANGE = 10
NRC = NRANGE // NC
RPT = RSIZE // NS
CAP = 3968
CHUNK = 128
SCH = 64

EPT = 15872
E_PAD = EPT * NW
CH = 992
NCH = EPT // CH

_mesh = plsc.VectorSubcoreMesh(core_axis_name="c", subcore_axis_name="s",
                               num_cores=NC, num_subcores=NS)
_f32 = jnp.float32
_i32 = jnp.int32


@functools.partial(
    pl.kernel,
    out_type=jax.ShapeDtypeStruct((NW * N_PAD,), _f32),
    mesh=_mesh,
    compiler_params=pltpu.CompilerParams(needs_layout_passes=False),
    scratch_types=[
        pltpu.VMEM((N_PAD,), _f32),
        pltpu.VMEM((CH,), _i32),
    ],
)
def _hist_sc(users_hbm, pos_hbm, out_hbm, hist_v, ids_v):
    c = lax.axis_index("c")
    s = lax.axis_index("s")
    p = c * NS + s
    zero16 = jnp.zeros((L,), _f32)
    ones16 = jnp.ones((L,), _f32)

    def _z(i, _):
        hist_v[pl.ds(i * L, L)] = zero16
        return 0
    lax.fori_loop(0, N_PAD // L, _z, 0)

    base = p * EPT

    def _chunk(k, _):
        off = base + k * CH

        def _scan(j, _):
            idx = ids_v[pl.ds(j * L, L)]
            plsc.addupdate_scatter(hist_v, [idx], ones16)
            return 0

        pltpu.sync_copy(users_hbm.at[pl.ds(off, CH)], ids_v)
        lax.fori_loop(0, CH // L, _scan, 0)
        pltpu.sync_copy(pos_hbm.at[pl.ds(off, CH)], ids_v)
        lax.fori_loop(0, CH // L, _scan, 0)
        return 0

    lax.fori_loop(0, NCH, _chunk, 0)

    pltpu.sync_copy(hist_v, out_hbm.at[pl.ds(p * N_PAD, N_PAD)])


@functools.partial(
    pl.kernel,
    out_type=(jax.ShapeDtypeStruct((NRANGE * NW * CAP,), _i32),
              jax.ShapeDtypeStruct((NRANGE * NW * CAP,), _i32),
              jax.ShapeDtypeStruct((NW * L,), _i32)),
    mesh=_mesh,
    compiler_params=pltpu.CompilerParams(needs_layout_passes=False),
    scratch_types=[
        pltpu.VMEM((NRANGE * CAP,), _i32),
        pltpu.VMEM((NRANGE * CAP,), _i32),
        pltpu.VMEM((CH,), _i32),
        pltpu.VMEM((CH,), _i32),
        pltpu.VMEM((L,), _i32),
    ],
)
def _bucket_sc(users_hbm, pos_hbm, bsrc_hbm, bdst_hbm, cnt_hbm,
               st_src, st_dst, u_v, p_v, cnt_v):
    c = lax.axis_index("c")
    s = lax.axis_index("s")
    p = c * NS + s
    lane = lax.iota(_i32, L)
    padv16 = jnp.full((L,), PADV, _i32)
    zero16 = jnp.zeros((L,), _i32)

    def _fill(i, _):
        st_src[pl.ds(i * L, L)] = padv16
        st_dst[pl.ds(i * L, L)] = zero16
        return 0
    lax.fori_loop(0, NRANGE * CAP // L, _fill, 0)

    base = p * EPT

    def _append(r, cr, src_vec, dst_vec, m):
        plsc.store_compressed(st_src.at[pl.ds(r * CAP + cr, L)], src_vec,
                              mask=m)
        plsc.store_compressed(st_dst.at[pl.ds(r * CAP + cr, L)],
                              dst_vec - r * RSIZE, mask=m)
        return jnp.minimum(cr + jnp.sum(m.astype(_i32)), CAP - L)

    def _chunk(k, counts):
        off = base + k * CH
        pltpu.sync_copy(users_hbm.at[pl.ds(off, CH)], u_v)
        pltpu.sync_copy(pos_hbm.at[pl.ds(off, CH)], p_v)

        def _scan(j, counts):
            counts = list(counts)
            uv = u_v[pl.ds(j * L, L)]
            pv = p_v[pl.ds(j * L, L)]
            for r in (4, 5, 6, 7, 8, 9):
                m = (pv >= r * RSIZE) & (pv < (r + 1) * RSIZE)
                counts[r] = _append(r, counts[r], uv, pv, m)
            for r in (0, 1, 2, 3, 4):
                m = (uv >= r * RSIZE) & (uv < (r + 1) * RSIZE)
                counts[r] = _append(r, counts[r], pv, uv, m)
            return tuple(counts)

        return lax.fori_loop(0, CH // L, _scan, counts)

    counts = lax.fori_loop(0, NCH, _chunk,
                           tuple(jnp.zeros((), _i32) for _ in range(NRANGE)))

    for r in range(NRANGE):
        pltpu.sync_copy(st_src.at[pl.ds(r * CAP, CAP)],
                        bsrc_hbm.at[pl.ds((r * NW + p) * CAP, CAP)])
        pltpu.sync_copy(st_dst.at[pl.ds(r * CAP, CAP)],
                        bdst_hbm.at[pl.ds((r * NW + p) * CAP, CAP)])

    cvec = jnp.zeros((L,), _i32)
    for r in range(NRANGE):
        cvec = jnp.where(lane == r, counts[r], cvec)
    cnt_v[pl.ds(0, L)] = cvec
    pltpu.sync_copy(cnt_v, cnt_hbm.at[pl.ds(p * L, L)])


CPB = CAP // SCH


@functools.partial(
    pl.kernel,
    out_type=jax.ShapeDtypeStruct((NRANGE * RSIZE, 2 * D), _f32),
    mesh=_mesh,
    compiler_params=pltpu.CompilerParams(needs_layout_passes=False,
                                         use_tc_tiling_on_sc=False),
    scratch_types=[
        pltpu.VMEM_SHARED((RSIZE, 2 * D), _f32),
        pltpu.VMEM((2 * CPB, SCH), _i32),
        pltpu.VMEM((2 * CPB, SCH), _i32),
        pltpu.VMEM((SCH, 2 * D), _f32),
        pltpu.VMEM((SCH, 2 * D), _f32),
        pltpu.VMEM((SCH, 2 * D), _f32),
        pltpu.VMEM((SCH, 2 * D), _f32),
        pltpu.VMEM((L,), _i32),
        pltpu.VMEM((L,), _i32),
        pltpu.SemaphoreType.DMA,
        pltpu.SemaphoreType.DMA,
        pltpu.SemaphoreType.DMA,
        pltpu.SemaphoreType.DMA,
    ],
)
def _spmm_sc(y_hbm, bsrc_hbm, bdst_hbm, cnt_hbm, s_hbm,
             acc_sh, sidx_v, didx_v, rows_0, rows_1, rows_2, rows_3,
             cra_v, crb_v, gsem, psem, zsem, ssem):
    c = lax.axis_index("c")
    s = lax.axis_index("s")
    lane = lax.iota(_i32, L)
    zero16 = jnp.zeros((L,), _f32)

    def _z(i, _):
        for jj in range(2 * D // L):
            rows_0[i, pl.ds(jj * L, L)] = zero16
        return 0

    pltpu.sync_copy(cnt_hbm.at[pl.ds((2 * s) * L, L)], cra_v)
    pltpu.sync_copy(cnt_hbm.at[pl.ds((2 * s + 1) * L, L)], crb_v)
    cra = cra_v[pl.ds(0, L)]
    crb = crb_v[pl.ds(0, L)]

    for j in range(NRC):
        r = 2 * j + c

        lax.fori_loop(0, SCH, _z, 0)

        def _zacc_args(i):
            return (rows_0, acc_sh.at[pl.ds(s * RPT + i * SCH, SCH)], zsem)
        for i in range(RPT // SCH):
            pltpu.async_copy(*_zacc_args(i))

        rowa = (r * NW + 2 * s) * CPB
        rowb = (r * NW + 2 * s + 1) * CPB
        pltpu.async_copy(bsrc_hbm.at[pl.ds(rowa, CPB)],
                         sidx_v.at[pl.ds(0, CPB)], psem)
        pltpu.async_copy(bsrc_hbm.at[pl.ds(rowb, CPB)],
                         sidx_v.at[pl.ds(CPB, CPB)], psem)
        pltpu.async_copy(bdst_hbm.at[pl.ds(rowa, CPB)],
                         didx_v.at[pl.ds(0, CPB)], psem)
        pltpu.async_copy(bdst_hbm.at[pl.ds(rowb, CPB)],
                         didx_v.at[pl.ds(CPB, CPB)], psem)

        ca = jnp.sum(jnp.where(lane == r, cra, 0))
        cb = jnp.sum(jnp.where(lane == r, crb, 0))
        nch_a = (ca + SCH - 1) // SCH
        nch = nch_a + (cb + SCH - 1) // SCH

        for i in range(RPT // SCH):
            pltpu.make_async_copy(*_zacc_args(i)).wait()
        pltpu.make_async_copy(bsrc_hbm.at[pl.ds(rowa, CPB)],
                              sidx_v.at[pl.ds(0, CPB)], psem).wait()
        pltpu.make_async_copy(bsrc_hbm.at[pl.ds(rowb, CPB)],
                              sidx_v.at[pl.ds(CPB, CPB)], psem).wait()
        pltpu.make_async_copy(bdst_hbm.at[pl.ds(rowa, CPB)],
                              didx_v.at[pl.ds(0, CPB)], psem).wait()
        pltpu.make_async_copy(bdst_hbm.at[pl.ds(rowb, CPB)],
                              didx_v.at[pl.ds(CPB, CPB)], psem).wait()
        plsc.subcore_barrier()

        def _row_of(k):
            return jnp.where(k < nch_a, k, k - nch_a + CPB)

        def _fire(k, buf):
            pltpu.async_copy(y_hbm.at[sidx_v.at[_row_of(k)]], buf, gsem)

        def _wait(k, buf):
            pltpu.make_async_copy(y_hbm.at[sidx_v.at[_row_of(k)]], buf,
                                  gsem).wait()

        def _scat(k, buf):
            pltpu.sync_copy(buf, acc_sh.at[didx_v.at[_row_of(k)]], add=True)

        def _wait_scat(buf):
            pass

        bufs = (rows_0, rows_1, rows_2, rows_3)
        for i in range(4):
            @pl.when(i < nch)
            def _(i=i):
                _fire(i, bufs[i])

        def _quad(q, _):
            k4 = 4 * q
            for i in range(4):
                k = k4 + i

                @pl.when(k < nch)
                def _(k=k, i=i):
                    _wait(k, bufs[i])
                    _scat(k, bufs[i])

                @pl.when(k + 4 < nch)
                def _(k=k, i=i):
                    _wait_scat(bufs[i])
                    _fire(k + 4, bufs[i])
            return 0

        lax.fori_loop(0, (nch + 3) // 4, _quad, 0)
        for i in range(4):
            @pl.when(i < jnp.minimum(nch, 4))
            def _(i=i):
                _wait_scat(bufs[i])

        plsc.subcore_barrier()
        pltpu.sync_copy(acc_sh.at[pl.ds(s * RPT, RPT)],
                        s_hbm.at[pl.ds(r * RSIZE + s * RPT, RPT)])
        plsc.subcore_barrier()


KPT = EPT // CHUNK
ERWS = E_PAD // CHUNK


@functools.partial(
    pl.kernel,
    out_type=(jax.ShapeDtypeStruct((ERWS, CHUNK), _f32),
              jax.ShapeDtypeStruct((ERWS, CHUNK), _f32)),
    mesh=_mesh,
    compiler_params=pltpu.CompilerParams(needs_layout_passes=False,
                                         use_tc_tiling_on_sc=False),
    scratch_types=[
        pltpu.VMEM((KPT, CHUNK), _i32),
        pltpu.VMEM((KPT, CHUNK), _i32),
        pltpu.VMEM((KPT, CHUNK), _i32),
        pltpu.VMEM((CHUNK, D), _f32),
        pltpu.VMEM((CHUNK, D), _f32),
        pltpu.VMEM((CHUNK, D), _f32),
        pltpu.VMEM((CHUNK, D), _f32),
        pltpu.VMEM((CHUNK, D), _f32),
        pltpu.VMEM((CHUNK, D), _f32),
        pltpu.VMEM((CHUNK, D), _f32),
        pltpu.VMEM((CHUNK, D), _f32),
        pltpu.VMEM((CHUNK, D), _f32),
        pltpu.VMEM((3, CHUNK), _f32),
        pltpu.VMEM((3, CHUNK), _f32),
        pltpu.SemaphoreType.DMA,
        pltpu.SemaphoreType.DMA,
        pltpu.SemaphoreType.DMA,
    ],
)
def _sim_sc(ef_hbm, u_hbm, p_hbm, n_hbm, psim_hbm, nsim_hbm,
            uid_v, pid_v, nid_v, ur_a, pr_a, nr_a, ur_b, pr_b, nr_b,
            ur_c, pr_c, nr_c, po_st, no_st, gsem, isem, osem):
    c = lax.axis_index("c")
    s = lax.axis_index("s")
    w = c * NS + s
    lane = lax.iota(_i32, L)
    rbase = w * KPT

    pltpu.async_copy(u_hbm.at[pl.ds(rbase, KPT)], uid_v, isem)
    pltpu.async_copy(p_hbm.at[pl.ds(rbase, KPT)], pid_v, isem)
    pltpu.async_copy(n_hbm.at[pl.ds(rbase, KPT)], nid_v, isem)
    pltpu.make_async_copy(u_hbm.at[pl.ds(rbase, KPT)], uid_v, isem).wait()
    pltpu.make_async_copy(p_hbm.at[pl.ds(rbase, KPT)], pid_v, isem).wait()
    pltpu.make_async_copy(n_hbm.at[pl.ds(rbase, KPT)], nid_v, isem).wait()

    bufs = ((ur_a, pr_a, nr_a), (ur_b, pr_b, nr_b), (ur_c, pr_c, nr_c))

    def _fire(k, ur, pr, nr):
        pltpu.async_copy(ef_hbm.at[uid_v.at[k]], ur, gsem)
        pltpu.async_copy(ef_hbm.at[pid_v.at[k]], pr, gsem)
        pltpu.async_copy(ef_hbm.at[nid_v.at[k]], nr, gsem)

    def _waitg(k, ur, pr, nr):
        pltpu.make_async_copy(ef_hbm.at[uid_v.at[k]], ur, gsem).wait()
        pltpu.make_async_copy(ef_hbm.at[pid_v.at[k]], pr, gsem).wait()
        pltpu.make_async_copy(ef_hbm.at[nid_v.at[k]], nr, gsem).wait()

    def _owrite(k, i):
        pltpu.async_copy(po_st.at[i], psim_hbm.at[rbase + k], osem)
        pltpu.async_copy(no_st.at[i], nsim_hbm.at[rbase + k], osem)

    def _owait(k, i):
        pltpu.make_async_copy(po_st.at[i], psim_hbm.at[rbase + k],
                              osem).wait()
        pltpu.make_async_copy(no_st.at[i], nsim_hbm.at[rbase + k],
                              osem).wait()

    def _compute(i, ur, pr, nr):
        for g in range(CHUNK // L):
            riv = lane + g * L

            def _dstep(t, carry):
                pacc, nacc = carry
                for dd in range(8):
                    col = jnp.full((L,), t * 8 + dd, _i32)
                    uv = plsc.load_gather(ur, [riv, col])
                    pv = plsc.load_gather(pr, [riv, col])
                    nv = plsc.load_gather(nr, [riv, col])
                    pacc = pacc + uv * pv
                    nacc = nacc + uv * nv
                return (pacc, nacc)

            pacc, nacc = lax.fori_loop(
                0, D // 8, _dstep,
                (jnp.zeros((L,), _f32), jnp.zeros((L,), _f32)))
            po_st[i, pl.ds(g * L, L)] = pacc * (1.0 / 16.0)
            no_st[i, pl.ds(g * L, L)] = nacc * (1.0 / 16.0)

    _fire(0, *bufs[0])
    _fire(1, *bufs[1])

    def _grp(q, _):
        for i in range(3):
            k = 3 * q + i

            @pl.when(k < KPT)
            def _(k=k, i=i):
                @pl.when(k + 2 < KPT)
                def _():
                    _fire(k + 2, *bufs[(i + 2) % 3])
                _waitg(k, *bufs[i])

                @pl.when(k >= 3)
                def _():
                    _owait(k - 3, i)
                _compute(i, *bufs[i])
                _owrite(k, i)
        return 0

    lax.fori_loop(0, (KPT + 2) // 3, _grp, 0)
    for i in range(3):
        _owait(KPT - 3 + i, (KPT - 3 + i) % 3)


def _norms(h):
    deg = jnp.sum(h, axis=1, keepdims=True)
    d1 = jnp.where(deg > 0, lax.rsqrt(jnp.maximum(deg, 1e-30)), 0.0)
    d2 = lax.rsqrt(deg + 1.0)
    return d1, d2


def _make_y(e, w1, w2, d1, d2):
    dn = (((1,), (1,)), ((), ()))
    x1 = lax.dot_general(e, w1, dn, preferred_element_type=_f32)
    x2 = lax.dot_general(e, w2, dn, preferred_element_type=_f32)
    return jnp.concatenate([d2 * x1, d1 * (e * x2)], axis=1)


def _layer_e(sv, yv, d1, d2):
    h = (d2 * (sv[:, :D] + yv[:, :D]) + d1 * (sv[:, D:] + yv[:, D:]))
    h = jnp.where(h >= 0, h, NEG_SLOPE * h)
    nr = jnp.sqrt(jnp.sum(h * h, axis=1, keepdims=True))
    return h / jnp.maximum(nr, 1e-12)


def _pre1_body(ht_ref, e_ref, w1_ref, w2_ref, y_ref, d1_ref, d2_ref):
    d1, d2 = _norms(ht_ref[...])
    d1_ref[...] = d1
    d2_ref[...] = d2
    y_ref[...] = _make_y(e_ref[...], w1_ref[...], w2_ref[...], d1, d2)


def _pre1_tc(hist_t, emb, w1, w2):
    return pl.pallas_call(
        _pre1_body,
        grid=(NBLK,),
        in_specs=[pl.BlockSpec((RB, NW), lambda i: (i, 0)),
                  pl.BlockSpec((RB, D), lambda i: (i, 0)),
                  pl.BlockSpec((D, D), lambda i: (0, 0)),
                  pl.BlockSpec((D, D), lambda i: (0, 0))],
        out_specs=(pl.BlockSpec((RB, 2 * D), lambda i: (i, 0)),
                   pl.BlockSpec((RB, 1), lambda i: (i, 0)),
                   pl.BlockSpec((RB, 1), lambda i: (i, 0))),
        out_shape=(jax.ShapeDtypeStruct((N_PAD, 2 * D), _f32),
                   jax.ShapeDtypeStruct((N_PAD, 1), _f32),
                   jax.ShapeDtypeStruct((N_PAD, 1), _f32)),
    )(hist_t, emb, w1, w2)


def _mid_body(s_ref, y_ref, d1_ref, d2_ref, a_ref, w1_ref, w2_ref,
              yn_ref, ao_ref):
    d1 = d1_ref[...]
    d2 = d2_ref[...]
    e = _layer_e(s_ref[...], y_ref[...], d1, d2)
    ao_ref[...] = a_ref[...] + e
    yn_ref[...] = _make_y(e, w1_ref[...], w2_ref[...], d1, d2)


def _mid_tc(s, y, d1, d2, acc, w1, w2):
    return pl.pallas_call(
        _mid_body,
        grid=(NBLK,),
        in_specs=[pl.BlockSpec((RB, 2 * D), lambda i: (i, 0)),
                  pl.BlockSpec((RB, 2 * D), lambda i: (i, 0)),
                  pl.BlockSpec((RB, 1), lambda i: (i, 0)),
                  pl.BlockSpec((RB, 1), lambda i: (i, 0)),
                  pl.BlockSpec((RB, D), lambda i: (i, 0)),
                  pl.BlockSpec((D, D), lambda i: (0, 0)),
                  pl.BlockSpec((D, D), lambda i: (0, 0))],
        out_specs=(pl.BlockSpec((RB, 2 * D), lambda i: (i, 0)),
                   pl.BlockSpec((RB, D), lambda i: (i, 0))),
        out_shape=(jax.ShapeDtypeStruct((N_PAD, 2 * D), _f32),
                   jax.ShapeDtypeStruct((N_PAD, D), _f32)),
    )(s, y, d1, d2, acc, w1, w2)


def _postf_body(s_ref, y_ref, d1_ref, d2_ref, a_ref, ao_ref):
    e = _layer_e(s_ref[...], y_ref[...], d1_ref[...], d2_ref[...])
    ao_ref[...] = a_ref[...] + e


def _postf_tc(s, y, d1, d2, acc):
    return pl.pallas_call(
        _postf_body,
        grid=(NBLK,),
        in_specs=[pl.BlockSpec((RB, 2 * D), lambda i: (i, 0)),
                  pl.BlockSpec((RB, 2 * D), lambda i: (i, 0)),
                  pl.BlockSpec((RB, 1), lambda i: (i, 0)),
                  pl.BlockSpec((RB, 1), lambda i: (i, 0)),
                  pl.BlockSpec((RB, D), lambda i: (i, 0))],
        out_specs=pl.BlockSpec((RB, D), lambda i: (i, 0)),
        out_shape=jax.ShapeDtypeStruct((N_PAD, D), _f32),
    )(s, y, d1, d2, acc)


def kernel(users, pos, neg, Emb, W1_1, W2_1, W1_2, W2_2, W1_3, W2_3):
    users_p = jnp.pad(users.astype(_i32), (0, E_PAD - E),
                      constant_values=PADV)
    pos_p = jnp.pad(pos.astype(_i32), (0, E_PAD - E), constant_values=PADV)
    neg_p = jnp.pad(neg.astype(_i32), (0, E_PAD - E), constant_values=PADV)
    emb_p = jnp.pad(Emb, ((0, N_PAD - N), (0, 0)))

    hist = _hist_sc(users_p, pos_p).reshape(NW, N_PAD)
    bsrc, bdst, cnt = _bucket_sc(users_p, pos_p)
    bsrc2 = bsrc.reshape(NRANGE * NW * CPB, SCH)
    bdst2 = bdst.reshape(NRANGE * NW * CPB, SCH)

    y1, d1, d2 = _pre1_tc(hist.T, emb_p, W1_1, W2_1)
    s1 = _spmm_sc(y1, bsrc2, bdst2, cnt)
    y2, acc1 = _mid_tc(s1, y1, d1, d2, emb_p, W1_2, W2_2)
    s2 = _spmm_sc(y2, bsrc2, bdst2, cnt)
    y3, acc2 = _mid_tc(s2, y2, d1, d2, acc1, W1_3, W2_3)
    s3 = _spmm_sc(y3, bsrc2, bdst2, cnt)
    acc = _postf_tc(s3, y3, d1, d2, acc2)

    psim, nsim = _sim_sc(acc, users_p.reshape(ERWS, CHUNK),
                         pos_p.reshape(ERWS, CHUNK),
                         neg_p.reshape(ERWS, CHUNK))
    return psim.reshape(E_PAD)[:E], nsim.reshape(E_PAD)[:E]
